# Initial kernel scaffold; baseline (speedup 1.0000x reference)
import jax
import jax.numpy as jnp
from jax.experimental import pallas as pl


def kernel(x, token_table, pos_table):
    raise NotImplementedError("write your pallas kernel here")


# speedup vs baseline: 1.3921x; 1.3921x over previous
import functools

import jax
import jax.numpy as jnp
from jax import lax
from jax.experimental import pallas as pl
from jax.experimental.pallas import tpu as pltpu
from jax.experimental.pallas import tpu_sc as plsc

MAXLEN = 200
EMBED = 32
NC = 2
NS = 16
NW = NC * NS
LANES = 16
HALVES = EMBED // LANES

CHUNK = 800
REPS = CHUNK // MAXLEN


def _make_kernel(total_rows: int):
  rows_per_w = total_rows // NW
  nchunks = rows_per_w // CHUNK
  mesh = plsc.VectorSubcoreMesh(core_axis_name="c", subcore_axis_name="s")

  @functools.partial(
      pl.kernel,
      mesh=mesh,
      out_type=jax.ShapeDtypeStruct((total_rows, EMBED), jnp.float32),
      compiler_params=pltpu.CompilerParams(use_tc_tiling_on_sc=False),
      scratch_types=[
          pltpu.VMEM((CHUNK,), jnp.int32),
          pltpu.VMEM((CHUNK, EMBED), jnp.float32),
          pltpu.VMEM((MAXLEN, EMBED), jnp.float32),
          pltpu.SemaphoreType.DMA,
      ],
  )
  def emb_kernel(x_hbm, tok_hbm, pos_hbm, out_hbm, idx_v, rows_v, pos_v, sem):
    wid = lax.axis_index("s") * NC + lax.axis_index("c")
    base0 = wid * rows_per_w
    pltpu.sync_copy(pos_hbm, pos_v)

    def chunk_body(ci, carry):
      base = base0 + ci * CHUNK
      pltpu.sync_copy(x_hbm.at[pl.ds(base, CHUNK)], idx_v)
      pltpu.async_copy(tok_hbm.at[idx_v], rows_v, sem).wait()

      def t_body(t, c2):
        for h in range(HALVES):
          pv = pos_v[t, pl.ds(h * LANES, LANES)]
          for rep in range(REPS):
            r = rep * MAXLEN + t
            rows_v[r, pl.ds(h * LANES, LANES)] = (
                rows_v[r, pl.ds(h * LANES, LANES)] + pv)
        return c2

      lax.fori_loop(0, MAXLEN, t_body, 0)
      pltpu.sync_copy(rows_v, out_hbm.at[pl.ds(base, CHUNK)])
      return carry

    lax.fori_loop(0, nchunks, chunk_body, 0)

  return emb_kernel


def kernel(x, token_table, pos_table):
  batch, maxlen = x.shape
  x_flat = x.reshape(-1).astype(jnp.int32)
  out = _make_kernel(x_flat.shape[0])(x_flat, token_table, pos_table)
  return out.reshape(batch, maxlen, EMBED)


# speedup vs baseline: 1.4834x; 1.0656x over previous
import functools

import jax
import jax.numpy as jnp
from jax import lax
from jax.experimental import pallas as pl
from jax.experimental.pallas import tpu as pltpu
from jax.experimental.pallas import tpu_sc as plsc

MAXLEN = 200
EMBED = 32
NC = 2
NS = 16
NW = NC * NS
LANES = 16

CHUNK = 800
REPS = CHUNK // MAXLEN


def _make_kernel(total_rows: int):
  rows_per_w = total_rows // NW
  nchunks = rows_per_w // CHUNK
  mesh = plsc.VectorSubcoreMesh(core_axis_name="c", subcore_axis_name="s")

  @functools.partial(
      pl.kernel,
      mesh=mesh,
      out_type=jax.ShapeDtypeStruct((total_rows, EMBED), jnp.float32),
      compiler_params=pltpu.CompilerParams(use_tc_tiling_on_sc=False),
      scratch_types=[
          pltpu.VMEM((rows_per_w,), jnp.int32),
          pltpu.VMEM((2, CHUNK, EMBED), jnp.float32),
          pltpu.VMEM((MAXLEN, EMBED), jnp.float32),
          pltpu.SemaphoreType.DMA,
          pltpu.SemaphoreType.DMA,
          pltpu.SemaphoreType.DMA,
          pltpu.SemaphoreType.DMA,
      ],
  )
  def emb_kernel(x_hbm, tok_hbm, pos_hbm, out_hbm,
                 idx_v, rows_v, pos_v, gsem0, gsem1, osem0, osem1):
    wid = lax.axis_index("s") * NC + lax.axis_index("c")
    base0 = wid * rows_per_w
    gsems = (gsem0, gsem1)
    osems = (osem0, osem1)

    pltpu.sync_copy(pos_hbm, pos_v)
    pltpu.sync_copy(x_hbm.at[pl.ds(base0, rows_per_w)], idx_v)

    def gather(ci, b, sem):
      pltpu.async_copy(
          tok_hbm.at[idx_v.at[pl.ds(ci * CHUNK, CHUNK)]], rows_v.at[b], sem)

    def out_slice(ci):
      return out_hbm.at[pl.ds(base0 + ci * CHUNK, CHUNK)]

    gather(0, 0, gsems[0])

    @pl.loop(0, nchunks, step=2)
    def chunk_loop(ci0):
      for b in range(2):
        nb = 1 - b
        ci = ci0 + b

        @pl.when(ci + 1 < nchunks)
        def _issue_next():
          @pl.when(ci >= 1)
          def _drain_prev_out():
            pltpu.make_async_copy(rows_v.at[nb], out_slice(0), osems[nb]).wait()
          gather(ci + 1, nb, gsems[nb])

        pltpu.make_async_copy(
            tok_hbm.at[idx_v.at[pl.ds(0, CHUNK)]], rows_v.at[b],
            gsems[b]).wait()

        @pl.loop(0, MAXLEN, unroll=2)
        def t_body(t):
          pv0 = pos_v[t, pl.ds(0, LANES)]
          pv1 = pos_v[t, pl.ds(LANES, LANES)]
          for rep in range(REPS):
            r = rep * MAXLEN + t
            rows_v[b, r, pl.ds(0, LANES)] = rows_v[b, r, pl.ds(0, LANES)] + pv0
            rows_v[b, r, pl.ds(LANES, LANES)] = (
                rows_v[b, r, pl.ds(LANES, LANES)] + pv1)

        pltpu.async_copy(rows_v.at[b], out_slice(ci), osems[b])

    pltpu.make_async_copy(rows_v.at[0], out_slice(0), osems[nchunks % 2]).wait()
    pltpu.make_async_copy(rows_v.at[1], out_slice(0),
                          osems[(nchunks + 1) % 2]).wait()

  return emb_kernel


def kernel(x, token_table, pos_table):
  batch, maxlen = x.shape
  x_flat = x.reshape(-1).astype(jnp.int32)
  out = _make_kernel(x_flat.shape[0])(x_flat, token_table, pos_table)
  return out.reshape(batch, maxlen, EMBED)
